# Initial kernel scaffold; baseline (speedup 1.0000x reference)
import jax
import jax.numpy as jnp
from jax.experimental import pallas as pl


def kernel(inputs, S, T, L, P, W, b):
    raise NotImplementedError("write your pallas kernel here")


# speedup vs baseline: 11.2354x; 11.2354x over previous
import functools

import jax
import jax.numpy as jnp
from jax import lax
from jax.experimental import pallas as pl
from jax.experimental.pallas import tpu as pltpu
from jax.experimental.pallas import tpu_sc as plsc

VOCAB = 1000000
D = 32
MAXLEN = 200
B = 4096

NC = 2
NS = 16
NW = NC * NS
NTOK = B * MAXLEN
ROWS_PER_W = NTOK // NW
DMA_ROWS = 128
NDMA = ROWS_PER_W // DMA_ROWS

ROW_BLK = 8000


def _matmul_body(s_ref, w_ref, b_ref, o_ref):
    o_ref[:] = (
        jnp.dot(s_ref[:], w_ref[:], preferred_element_type=jnp.float32)
        + b_ref[:]
    )


def _table_times_w(S, W, b):
    return pl.pallas_call(
        _matmul_body,
        grid=(VOCAB // ROW_BLK,),
        in_specs=[
            pl.BlockSpec((ROW_BLK, D), lambda i: (i, 0)),
            pl.BlockSpec((D, 2), lambda i: (0, 0)),
            pl.BlockSpec((1, 2), lambda i: (0, 0)),
        ],
        out_specs=pl.BlockSpec((ROW_BLK, 2), lambda i: (i, 0)),
        out_shape=jax.ShapeDtypeStruct((VOCAB, 2), jnp.float32),
    )(S, W, b.reshape(1, 2))


PH = 5
KP = NDMA // PH


def _gather_body(sw_hbm, idx_hbm, out_hbm, idx_v, buf0, buf1, sem_g0, sem_g1, sem_o):
    wid = lax.axis_index("s") * NC + lax.axis_index("c")
    pltpu.sync_copy(idx_hbm.at[wid], idx_v)
    bufs = (buf0, buf1)
    sems = (sem_g0, sem_g1)

    def fire_phase(p, buf, sem):
        def fire(j, c):
            pltpu.make_async_copy(
                sw_hbm.at[idx_v.at[p * KP + j]], buf.at[j], sem
            ).start()
            return c

        lax.fori_loop(0, KP, fire, 0)

    def drain_phase(buf, sem):
        def dr(j, c):
            pltpu.make_async_copy(
                sw_hbm.at[idx_v.at[0]], buf.at[j], sem
            ).wait()
            return c

        lax.fori_loop(0, KP, dr, 0)

    def out_copy(p, buf):
        return pltpu.make_async_copy(
            buf, out_hbm.at[wid].at[pl.ds(p * KP, KP)], sem_o
        )

    fire_phase(0, buf0, sem_g0)
    fire_phase(1, buf1, sem_g1)
    for p in range(PH):
        buf, sem = bufs[p % 2], sems[p % 2]
        drain_phase(buf, sem)
        out_copy(p, buf).start()
        if p + 2 < PH:
            out_copy(p, buf).wait()
            fire_phase(p + 2, buf, sem)
    out_copy(PH - 2, bufs[(PH - 2) % 2]).wait()
    out_copy(PH - 1, bufs[(PH - 1) % 2]).wait()


def _sc_gather(sw, idx3):
    mesh = plsc.VectorSubcoreMesh(core_axis_name="c", subcore_axis_name="s")
    f = functools.partial(
        pl.kernel,
        mesh=mesh,
        out_type=jax.ShapeDtypeStruct((NW, NDMA, DMA_ROWS, 2), jnp.float32),
        scratch_types=[
            pltpu.VMEM((NDMA, DMA_ROWS), jnp.int32),
            pltpu.VMEM((KP, DMA_ROWS, 2), jnp.float32),
            pltpu.VMEM((KP, DMA_ROWS, 2), jnp.float32),
            pltpu.SemaphoreType.DMA,
            pltpu.SemaphoreType.DMA,
            pltpu.SemaphoreType.DMA,
        ],
        compiler_params=pltpu.CompilerParams(use_tc_tiling_on_sc=False),
    )(_gather_body)
    return f(sw, idx3)


def _softmax_body(x_ref, o_ref):
    x = x_ref[:]
    m = jnp.max(x, axis=0, keepdims=True)
    e = jnp.exp(x - m)
    o_ref[:] = e / jnp.sum(e, axis=0, keepdims=True)


def _softmax0(x):
    return pl.pallas_call(
        _softmax_body,
        out_shape=jax.ShapeDtypeStruct(x.shape, jnp.float32),
    )(x)


def kernel(inputs, S, T, L, P, W, b):
    idx3 = inputs[0].astype(jnp.int32).reshape(NW, NDMA, DMA_ROWS)
    sw = _table_times_w(S, W.astype(jnp.float32), b.astype(jnp.float32))
    logits = _sc_gather(sw, idx3)
    y = _softmax0(logits.reshape(B, MAXLEN * 2))
    return y.reshape(B, MAXLEN, 2)
